# Initial kernel scaffold; baseline (speedup 1.0000x reference)
import functools
import jax
import jax.numpy as jnp
from jax import lax
from jax.experimental import pallas as pl
from jax.experimental.pallas import tpu as pltpu
from jax.experimental.pallas import tpu_sc as plsc

N = 10000
E = 320000
D = 128
NPAD = 10240
NC = 2
NS = 16
NW = NC * NS
EPT = E // NW
CH = 2000
NCH = EPT // CH
SPT = NPAD // NS
BLK = 1024
GRID = NPAD // BLK

_mesh = plsc.VectorSubcoreMesh(core_axis_name="c", subcore_axis_name="s")
_f32 = jnp.float32


@functools.partial(
    pl.kernel,
    out_type=jax.ShapeDtypeStruct((2, NC, NPAD), _f32),
    mesh=_mesh,
    scratch_types=[
        pltpu.VMEM((CH,), jnp.int32),
        pltpu.VMEM((CH,), _f32),
        pltpu.VMEM_SHARED((NPAD,), _f32),
        pltpu.VMEM_SHARED((NPAD,), _f32),
    ],
)
def _sc_hist(dst_p, dst_l, ones_hbm, z1_hbm, hist_out, idx_v, ones_v, acc_p, acc_l):
    c = lax.axis_index("c")
    s = lax.axis_index("s")
    sl = pl.ds(s * SPT, SPT)
    pltpu.sync_copy(z1_hbm.at[sl], acc_p.at[sl])
    pltpu.sync_copy(z1_hbm.at[sl], acc_l.at[sl])
    pltpu.sync_copy(ones_hbm, ones_v)
    plsc.subcore_barrier()
    base = (s * NC + c) * EPT
    for dref, acc in ((dst_p, acc_p), (dst_l, acc_l)):
        for k in range(NCH):
            pltpu.sync_copy(dref.at[pl.ds(base + k * CH, CH)], idx_v)
            pltpu.sync_copy(ones_v, acc.at[idx_v], add=True)
    plsc.subcore_barrier()
    pltpu.sync_copy(acc_p.at[sl], hist_out.at[0].at[c].at[sl])
    pltpu.sync_copy(acc_l.at[sl], hist_out.at[1].at[c].at[sl])


def _tc1_body(hist_ref, xp_ref, xl_ref, wp_ref, wl_ref, hp_ref, hl_ref, dinv_ref):
    hist = hist_ref[...]
    deg = hist[:, 0] + hist[:, 1] + 1.0
    dinv = lax.rsqrt(deg)
    dinv_ref[...] = dinv
    hp_ref[...] = jnp.dot(xp_ref[...], wp_ref[...],
                          preferred_element_type=_f32) * dinv[0]
    hl_ref[...] = jnp.dot(xl_ref[...], wl_ref[...],
                          preferred_element_type=_f32) * dinv[1]


def _tc1(hist4, xp, xl, wp, wl):
    return pl.pallas_call(
        _tc1_body,
        grid=(GRID,),
        in_specs=[
            pl.BlockSpec((2, NC, BLK, 1), lambda i: (0, 0, i, 0)),
            pl.BlockSpec((BLK, D), lambda i: (i, 0)),
            pl.BlockSpec((BLK, D), lambda i: (i, 0)),
            pl.BlockSpec((D, 16), lambda i: (0, 0)),
            pl.BlockSpec((D, 16), lambda i: (0, 0)),
        ],
        out_specs=[
            pl.BlockSpec((BLK, 16), lambda i: (i, 0)),
            pl.BlockSpec((BLK, 16), lambda i: (i, 0)),
            pl.BlockSpec((2, BLK, 1), lambda i: (0, i, 0)),
        ],
        out_shape=[
            jax.ShapeDtypeStruct((NPAD, 16), _f32),
            jax.ShapeDtypeStruct((NPAD, 16), _f32),
            jax.ShapeDtypeStruct((2, NPAD, 1), _f32),
        ],
    )(hist4, xp, xl, wp, wl)


@functools.partial(
    pl.kernel,
    out_type=[
        jax.ShapeDtypeStruct((NC, NPAD, 16), _f32),
        jax.ShapeDtypeStruct((NC, NPAD, 16), _f32),
        jax.ShapeDtypeStruct((NC, NPAD), _f32),
        jax.ShapeDtypeStruct((NC, NPAD), _f32),
    ],
    mesh=_mesh,
    scratch_types=[
        pltpu.VMEM((CH,), jnp.int32),
        pltpu.VMEM((CH,), jnp.int32),
        pltpu.VMEM((CH, 16), _f32),
        pltpu.VMEM((CH,), _f32),
        pltpu.VMEM_SHARED((NPAD, 16), _f32),
        pltpu.VMEM_SHARED((NPAD, 16), _f32),
        pltpu.VMEM_SHARED((NPAD,), _f32),
        pltpu.VMEM_SHARED((NPAD,), _f32),
    ],
)
def _sc_edges(src_p, dst_p, src_l, dst_l, hp, hl, dvp, dvl, z16_hbm, z1_hbm,
              rop, rol, cop, col_,
              sidx, didx, rows, vals, racc_p, racc_l, cacc_p, cacc_l):
    c = lax.axis_index("c")
    s = lax.axis_index("s")
    sl = pl.ds(s * SPT, SPT)
    pltpu.sync_copy(z16_hbm.at[sl], racc_p.at[sl])
    pltpu.sync_copy(z16_hbm.at[sl], racc_l.at[sl])
    pltpu.sync_copy(z1_hbm.at[sl], cacc_p.at[sl])
    pltpu.sync_copy(z1_hbm.at[sl], cacc_l.at[sl])
    plsc.subcore_barrier()
    base = (s * NC + c) * EPT
    for sref, dref, href, dvref, racc, cacc in (
        (src_p, dst_p, hp, dvp, racc_p, cacc_p),
        (src_l, dst_l, hl, dvl, racc_l, cacc_l),
    ):
        for k in range(NCH):
            off = base + k * CH
            pltpu.sync_copy(sref.at[pl.ds(off, CH)], sidx)
            pltpu.sync_copy(dref.at[pl.ds(off, CH)], didx)
            pltpu.sync_copy(href.at[sidx], rows)
            pltpu.sync_copy(rows, racc.at[didx], add=True)
            pltpu.sync_copy(dvref.at[didx], vals)
            pltpu.sync_copy(vals, cacc.at[sidx], add=True)
    plsc.subcore_barrier()
    pltpu.sync_copy(racc_p.at[sl], rop.at[c].at[sl])
    pltpu.sync_copy(racc_l.at[sl], rol.at[c].at[sl])
    pltpu.sync_copy(cacc_p.at[sl], cop.at[c].at[sl])
    pltpu.sync_copy(cacc_l.at[sl], col_.at[c].at[sl])


def _tc2_body(rop_ref, rol_ref, cop_ref, col_ref, hp_ref, hl_ref, dinv_ref,
              bpin_ref, blin_ref, wpo_ref, wlo_ref, bpo_ref, blo_ref,
              w1_ref, b1_ref, w2_ref, b2_ref, w3_ref, b3_ref, act_ref,
              out_ref, sacc):
    i = pl.program_id(0)

    @pl.when(i == 0)
    def _():
        sacc[...] = jnp.zeros((8, 128), _f32)

    gid = lax.broadcasted_iota(jnp.int32, (BLK, 1), 0) + i * BLK
    valid = (gid < N).astype(_f32)

    for g, (ro_ref, co_ref, h_ref, b_ref) in enumerate((
        (rop_ref, cop_ref, hp_ref, bpin_ref),
        (rol_ref, col_ref, hl_ref, blin_ref),
    )):
        ro = ro_ref[...]
        dinv = dinv_ref[...][g]
        acc = ro[0] + ro[1] + h_ref[...]
        p1 = jax.nn.relu(dinv * acc + b_ref[...][None, :])
        co = co_ref[...]
        cvec = dinv * (co[0] + co[1] + dinv) * valid
        sacc[g, 0:16] += jnp.sum(cvec * p1, axis=0)

    @pl.when(i == GRID - 1)
    def _():
        inv_n = 1.0 / N
        sp = sacc[0:1, 0:16] * inv_n
        sl_ = sacc[1:2, 0:16] * inv_n
        pooled_p = jnp.dot(sp, wpo_ref[...], preferred_element_type=_f32) \
            + bpo_ref[...][None, :]
        pooled_l = jnp.dot(sl_, wlo_ref[...], preferred_element_type=_f32) \
            + blo_ref[...][None, :]
        m = jnp.concatenate([pooled_p, pooled_l], axis=1)
        fp = jax.nn.relu(jnp.dot(m, w1_ref[...],
                                 preferred_element_type=_f32)
                         + b1_ref[...][None, :])
        h2 = jnp.concatenate([fp, act_ref[...]], axis=1)
        pol = jax.nn.relu(jnp.dot(h2, w2_ref[...],
                                  preferred_element_type=_f32)
                          + b2_ref[...][None, :])
        out_ref[...] = jnp.dot(pol, w3_ref[...],
                               preferred_element_type=_f32) \
            + b3_ref[...][None, :]


def _tc2(rop, rol, cop4, col4, hp, hl, dinv, b_pin, b_lin, W_pout, W_lout,
         b_pout, b_lout, W1, b1, W2, b2, W3, b3, action):
    def full(shape):
        return pl.BlockSpec(shape, lambda *_: tuple(0 for _ in shape))
    return pl.pallas_call(
        _tc2_body,
        grid=(GRID,),
        in_specs=[
            pl.BlockSpec((NC, BLK, 16), lambda i: (0, i, 0)),
            pl.BlockSpec((NC, BLK, 16), lambda i: (0, i, 0)),
            pl.BlockSpec((NC, BLK, 1), lambda i: (0, i, 0)),
            pl.BlockSpec((NC, BLK, 1), lambda i: (0, i, 0)),
            pl.BlockSpec((BLK, 16), lambda i: (i, 0)),
            pl.BlockSpec((BLK, 16), lambda i: (i, 0)),
            pl.BlockSpec((2, BLK, 1), lambda i: (0, i, 0)),
            full((16,)), full((16,)),
            full((16, 50)), full((16, 50)),
            full((50,)), full((50,)),
            full((100, 60)), full((60,)),
            full((100, 10)), full((10,)),
            full((10, 1)), full((1,)),
            full((1, 40)),
        ],
        out_specs=pl.BlockSpec((1, 1), lambda i: (0, 0)),
        out_shape=jax.ShapeDtypeStruct((1, 1), _f32),
        scratch_shapes=[pltpu.VMEM((8, 128), _f32)],
    )(rop, rol, cop4, col4, hp, hl, dinv, b_pin, b_lin, W_pout, W_lout,
      b_pout, b_lout, W1, b1, W2, b2, W3, b3, action)


def kernel(protein_x, protein_edge_index, ligand_x, ligand_edge_index, action,
           W_pin, b_pin, W_pout, b_pout, W_lin, b_lin, W_lout, b_lout,
           W1, b1, W2, b2, W3, b3):
    z1 = jnp.zeros((NPAD,), _f32)
    z16 = jnp.zeros((NPAD, 16), _f32)
    ones = jnp.ones((CH,), _f32)
    xp = jnp.pad(protein_x, ((0, NPAD - N), (0, 0)))
    xl = jnp.pad(ligand_x, ((0, NPAD - N), (0, 0)))
    src_p, dst_p = protein_edge_index[0], protein_edge_index[1]
    src_l, dst_l = ligand_edge_index[0], ligand_edge_index[1]

    hist = _sc_hist(dst_p, dst_l, ones, z1)
    hp, hl, dinv = _tc1(hist.reshape(2, NC, NPAD, 1), xp, xl, W_pin, W_lin)
    dvp = dinv[0].reshape(NPAD)
    dvl = dinv[1].reshape(NPAD)
    rop, rol, cop, col_ = _sc_edges(src_p, dst_p, src_l, dst_l,
                                    hp, hl, dvp, dvl, z16, z1)
    return _tc2(rop, rol, cop.reshape(NC, NPAD, 1), col_.reshape(NC, NPAD, 1),
                hp, hl, dinv, b_pin, b_lin, W_pout, W_lout, b_pout, b_lout,
                W1, b1, W2, b2, W3, b3, action)


# speedup vs baseline: 50.3964x; 50.3964x over previous
import functools
import jax
import jax.numpy as jnp
from jax import lax
from jax.experimental import pallas as pl
from jax.experimental.pallas import tpu as pltpu
from jax.experimental.pallas import tpu_sc as plsc

N = 10000
E = 320000
D = 128
NPAD = 10240
NC = 2
NS = 16
NW = NC * NS
EPT = E // NW
CH = 2000
NCH = EPT // CH
SPT = NPAD // NS
BLK = 1024
GRID = NPAD // BLK

_mesh = plsc.VectorSubcoreMesh(core_axis_name="c", subcore_axis_name="s")
_f32 = jnp.float32
_sc_params = pltpu.CompilerParams(use_tc_tiling_on_sc=False)


@functools.partial(
    pl.kernel,
    out_type=jax.ShapeDtypeStruct((2, NC, NPAD), _f32),
    mesh=_mesh,
    scratch_types=[
        pltpu.VMEM((CH,), jnp.int32),
        pltpu.VMEM((CH,), _f32),
        pltpu.VMEM_SHARED((NPAD,), _f32),
        pltpu.VMEM_SHARED((NPAD,), _f32),
    ],
    compiler_params=_sc_params,
)
def _sc_hist(dst_p, dst_l, ones_hbm, z1_hbm, hist_out, idx_v, ones_v, acc_p, acc_l):
    c = lax.axis_index("c")
    s = lax.axis_index("s")
    sl = pl.ds(s * SPT, SPT)
    pltpu.sync_copy(z1_hbm.at[sl], acc_p.at[sl])
    pltpu.sync_copy(z1_hbm.at[sl], acc_l.at[sl])
    pltpu.sync_copy(ones_hbm, ones_v)
    plsc.subcore_barrier()
    base = (s * NC + c) * EPT
    for dref, acc in ((dst_p, acc_p), (dst_l, acc_l)):
        for k in range(NCH):
            pltpu.sync_copy(dref.at[pl.ds(base + k * CH, CH)], idx_v)
            pltpu.sync_copy(ones_v, acc.at[idx_v], add=True)
    plsc.subcore_barrier()
    pltpu.sync_copy(acc_p.at[sl], hist_out.at[0].at[c].at[sl])
    pltpu.sync_copy(acc_l.at[sl], hist_out.at[1].at[c].at[sl])


def _tc1_body(hist_ref, xp_ref, xl_ref, wp_ref, wl_ref, hp_ref, hl_ref, dinv_ref):
    hist = hist_ref[...]
    deg = hist[:, 0] + hist[:, 1] + 1.0
    dinv = lax.rsqrt(deg)
    dinv_ref[...] = dinv
    hp_ref[...] = jnp.dot(xp_ref[...], wp_ref[...],
                          preferred_element_type=_f32) * dinv[0]
    hl_ref[...] = jnp.dot(xl_ref[...], wl_ref[...],
                          preferred_element_type=_f32) * dinv[1]


def _tc1(hist4, xp, xl, wp, wl):
    return pl.pallas_call(
        _tc1_body,
        grid=(GRID,),
        in_specs=[
            pl.BlockSpec((2, NC, BLK, 1), lambda i: (0, 0, i, 0)),
            pl.BlockSpec((BLK, D), lambda i: (i, 0)),
            pl.BlockSpec((BLK, D), lambda i: (i, 0)),
            pl.BlockSpec((D, 16), lambda i: (0, 0)),
            pl.BlockSpec((D, 16), lambda i: (0, 0)),
        ],
        out_specs=[
            pl.BlockSpec((BLK, 16), lambda i: (i, 0)),
            pl.BlockSpec((BLK, 16), lambda i: (i, 0)),
            pl.BlockSpec((2, BLK, 1), lambda i: (0, i, 0)),
        ],
        out_shape=[
            jax.ShapeDtypeStruct((NPAD, 16), _f32),
            jax.ShapeDtypeStruct((NPAD, 16), _f32),
            jax.ShapeDtypeStruct((2, NPAD, 1), _f32),
        ],
    )(hist4, xp, xl, wp, wl)


@functools.partial(
    pl.kernel,
    out_type=[
        jax.ShapeDtypeStruct((NC, NPAD, 16), _f32),
        jax.ShapeDtypeStruct((NC, NPAD, 16), _f32),
        jax.ShapeDtypeStruct((NC, NPAD), _f32),
        jax.ShapeDtypeStruct((NC, NPAD), _f32),
    ],
    mesh=_mesh,
    scratch_types=[
        pltpu.VMEM((CH,), jnp.int32),
        pltpu.VMEM((CH,), jnp.int32),
        pltpu.VMEM((CH, 16), _f32),
        pltpu.VMEM((CH,), _f32),
        pltpu.VMEM_SHARED((NPAD, 16), _f32),
        pltpu.VMEM_SHARED((NPAD, 16), _f32),
        pltpu.VMEM_SHARED((NPAD,), _f32),
        pltpu.VMEM_SHARED((NPAD,), _f32),
    ],
    compiler_params=_sc_params,
)
def _sc_edges(src_p, dst_p, src_l, dst_l, hp, hl, dvp, dvl, z16_hbm, z1_hbm,
              rop, rol, cop, col_,
              sidx, didx, rows, vals, racc_p, racc_l, cacc_p, cacc_l):
    c = lax.axis_index("c")
    s = lax.axis_index("s")
    sl = pl.ds(s * SPT, SPT)
    pltpu.sync_copy(z16_hbm.at[sl], racc_p.at[sl])
    pltpu.sync_copy(z16_hbm.at[sl], racc_l.at[sl])
    pltpu.sync_copy(z1_hbm.at[sl], cacc_p.at[sl])
    pltpu.sync_copy(z1_hbm.at[sl], cacc_l.at[sl])
    plsc.subcore_barrier()
    base = (s * NC + c) * EPT
    for sref, dref, href, dvref, racc, cacc in (
        (src_p, dst_p, hp, dvp, racc_p, cacc_p),
        (src_l, dst_l, hl, dvl, racc_l, cacc_l),
    ):
        for k in range(NCH):
            off = base + k * CH
            pltpu.sync_copy(sref.at[pl.ds(off, CH)], sidx)
            pltpu.sync_copy(dref.at[pl.ds(off, CH)], didx)
            pltpu.sync_copy(href.at[sidx], rows)
            pltpu.sync_copy(rows, racc.at[didx], add=True)
            pltpu.sync_copy(dvref.at[didx], vals)
            pltpu.sync_copy(vals, cacc.at[sidx], add=True)
    plsc.subcore_barrier()
    pltpu.sync_copy(racc_p.at[sl], rop.at[c].at[sl])
    pltpu.sync_copy(racc_l.at[sl], rol.at[c].at[sl])
    pltpu.sync_copy(cacc_p.at[sl], cop.at[c].at[sl])
    pltpu.sync_copy(cacc_l.at[sl], col_.at[c].at[sl])


def _tc2_body(rop_ref, rol_ref, cop_ref, col_ref, hp_ref, hl_ref, dinv_ref,
              bpin_ref, blin_ref, wpo_ref, wlo_ref, bpo_ref, blo_ref,
              w1_ref, b1_ref, w2_ref, b2_ref, w3_ref, b3_ref, act_ref,
              out_ref, sacc):
    i = pl.program_id(0)

    @pl.when(i == 0)
    def _():
        sacc[...] = jnp.zeros((8, 128), _f32)

    gid = lax.broadcasted_iota(jnp.int32, (BLK, 1), 0) + i * BLK
    valid = (gid < N).astype(_f32)

    for g, (ro_ref, co_ref, h_ref, b_ref) in enumerate((
        (rop_ref, cop_ref, hp_ref, bpin_ref),
        (rol_ref, col_ref, hl_ref, blin_ref),
    )):
        ro = ro_ref[...]
        dinv = dinv_ref[...][g]
        acc = ro[0] + ro[1] + h_ref[...]
        p1 = jax.nn.relu(dinv * acc + b_ref[...][None, :])
        co = co_ref[...]
        cvec = dinv * (co[0] + co[1] + dinv) * valid
        sacc[g, 0:16] += jnp.sum(cvec * p1, axis=0)

    @pl.when(i == GRID - 1)
    def _():
        inv_n = 1.0 / N
        sp = sacc[0:1, 0:16] * inv_n
        sl_ = sacc[1:2, 0:16] * inv_n
        pooled_p = jnp.dot(sp, wpo_ref[...], preferred_element_type=_f32) \
            + bpo_ref[...][None, :]
        pooled_l = jnp.dot(sl_, wlo_ref[...], preferred_element_type=_f32) \
            + blo_ref[...][None, :]
        m = jnp.concatenate([pooled_p, pooled_l], axis=1)
        fp = jax.nn.relu(jnp.dot(m, w1_ref[...],
                                 preferred_element_type=_f32)
                         + b1_ref[...][None, :])
        h2 = jnp.concatenate([fp, act_ref[...]], axis=1)
        pol = jax.nn.relu(jnp.dot(h2, w2_ref[...],
                                  preferred_element_type=_f32)
                          + b2_ref[...][None, :])
        out_ref[...] = jnp.dot(pol, w3_ref[...],
                               preferred_element_type=_f32) \
            + b3_ref[...][None, :]


def _tc2(rop, rol, cop4, col4, hp, hl, dinv, b_pin, b_lin, W_pout, W_lout,
         b_pout, b_lout, W1, b1, W2, b2, W3, b3, action):
    def full(shape):
        return pl.BlockSpec(shape, lambda *_: tuple(0 for _ in shape))
    return pl.pallas_call(
        _tc2_body,
        grid=(GRID,),
        in_specs=[
            pl.BlockSpec((NC, BLK, 16), lambda i: (0, i, 0)),
            pl.BlockSpec((NC, BLK, 16), lambda i: (0, i, 0)),
            pl.BlockSpec((NC, BLK, 1), lambda i: (0, i, 0)),
            pl.BlockSpec((NC, BLK, 1), lambda i: (0, i, 0)),
            pl.BlockSpec((BLK, 16), lambda i: (i, 0)),
            pl.BlockSpec((BLK, 16), lambda i: (i, 0)),
            pl.BlockSpec((2, BLK, 1), lambda i: (0, i, 0)),
            full((16,)), full((16,)),
            full((16, 50)), full((16, 50)),
            full((50,)), full((50,)),
            full((100, 60)), full((60,)),
            full((100, 10)), full((10,)),
            full((10, 1)), full((1,)),
            full((1, 40)),
        ],
        out_specs=pl.BlockSpec((1, 1), lambda i: (0, 0)),
        out_shape=jax.ShapeDtypeStruct((1, 1), _f32),
        scratch_shapes=[pltpu.VMEM((8, 128), _f32)],
    )(rop, rol, cop4, col4, hp, hl, dinv, b_pin, b_lin, W_pout, W_lout,
      b_pout, b_lout, W1, b1, W2, b2, W3, b3, action)


def kernel(protein_x, protein_edge_index, ligand_x, ligand_edge_index, action,
           W_pin, b_pin, W_pout, b_pout, W_lin, b_lin, W_lout, b_lout,
           W1, b1, W2, b2, W3, b3):
    z1 = jnp.zeros((NPAD,), _f32)
    z16 = jnp.zeros((NPAD, 16), _f32)
    ones = jnp.ones((CH,), _f32)
    xp = jnp.pad(protein_x, ((0, NPAD - N), (0, 0)))
    xl = jnp.pad(ligand_x, ((0, NPAD - N), (0, 0)))
    src_p, dst_p = protein_edge_index[0], protein_edge_index[1]
    src_l, dst_l = ligand_edge_index[0], ligand_edge_index[1]

    hist = _sc_hist(dst_p, dst_l, ones, z1)
    hp, hl, dinv = _tc1(hist.reshape(2, NC, NPAD, 1), xp, xl, W_pin, W_lin)
    dvp = dinv[0].reshape(NPAD)
    dvl = dinv[1].reshape(NPAD)
    rop, rol, cop, col_ = _sc_edges(src_p, dst_p, src_l, dst_l,
                                    hp, hl, dvp, dvl, z16, z1)
    return _tc2(rop, rol, cop.reshape(NC, NPAD, 1), col_.reshape(NC, NPAD, 1),
                hp, hl, dinv, b_pin, b_lin, W_pout, W_lout, b_pout, b_lout,
                W1, b1, W2, b2, W3, b3, action)


# speedup vs baseline: 65.6489x; 1.3027x over previous
import functools
import jax
import jax.numpy as jnp
from jax import lax
from jax.experimental import pallas as pl
from jax.experimental.pallas import tpu as pltpu
from jax.experimental.pallas import tpu_sc as plsc

N = 10000
E = 320000
D = 128
NPAD = 10240
NC = 2
NS = 16
NW = NC * NS
EPT = E // NW
CH = 2000
NCH = EPT // CH
SPT = NPAD // NS
BLK = 1024
GRID = NPAD // BLK

_mesh = plsc.VectorSubcoreMesh(core_axis_name="c", subcore_axis_name="s")
_f32 = jnp.float32
_sc_params = pltpu.CompilerParams(use_tc_tiling_on_sc=False,
                                  needs_layout_passes=False)


@functools.partial(
    pl.kernel,
    out_type=jax.ShapeDtypeStruct((2, NC, NPAD), _f32),
    mesh=_mesh,
    scratch_types=[
        pltpu.VMEM((NCH, CH), jnp.int32),
        pltpu.VMEM((CH,), _f32),
        pltpu.VMEM_SHARED((NPAD,), _f32),
        pltpu.VMEM_SHARED((NPAD,), _f32),
        pltpu.SemaphoreType.DMA,
    ],
    compiler_params=_sc_params,
)
def _sc_hist(dst_p, dst_l, ones_hbm, z1_hbm, hist_out, idx2d, ones_v, acc_p,
             acc_l, sem):
    c = lax.axis_index("c")
    s = lax.axis_index("s")
    sl = pl.ds(s * SPT, SPT)
    pltpu.sync_copy(z1_hbm.at[sl], acc_p.at[sl])
    pltpu.sync_copy(z1_hbm.at[sl], acc_l.at[sl])
    pltpu.sync_copy(ones_hbm, ones_v)
    plsc.subcore_barrier()
    base = (s * NC + c) * EPT
    for dref, acc in ((dst_p, acc_p), (dst_l, acc_l)):
        descs = [
            pltpu.async_copy(dref.at[pl.ds(base + k * CH, CH)], idx2d.at[k], sem)
            for k in range(NCH)
        ]
        for d in descs:
            d.wait()
        for k in range(NCH):
            pltpu.sync_copy(ones_v, acc.at[idx2d.at[k]], add=True)
    plsc.subcore_barrier()
    pltpu.sync_copy(acc_p.at[sl], hist_out.at[0].at[c].at[sl])
    pltpu.sync_copy(acc_l.at[sl], hist_out.at[1].at[c].at[sl])


def _tc1_body(hist_ref, xp_ref, xl_ref, wp_ref, wl_ref, hp_ref, hl_ref, dinv_ref):
    hist = hist_ref[...]
    deg = hist[:, 0] + hist[:, 1] + 1.0
    dinv = lax.rsqrt(deg)
    dinv_ref[...] = dinv
    hp_ref[...] = jnp.dot(xp_ref[...], wp_ref[...],
                          preferred_element_type=_f32) * dinv[0]
    hl_ref[...] = jnp.dot(xl_ref[...], wl_ref[...],
                          preferred_element_type=_f32) * dinv[1]


def _tc1(hist4, xp, xl, wp, wl):
    return pl.pallas_call(
        _tc1_body,
        grid=(GRID,),
        in_specs=[
            pl.BlockSpec((2, NC, BLK, 1), lambda i: (0, 0, i, 0)),
            pl.BlockSpec((BLK, D), lambda i: (i, 0)),
            pl.BlockSpec((BLK, D), lambda i: (i, 0)),
            pl.BlockSpec((D, 16), lambda i: (0, 0)),
            pl.BlockSpec((D, 16), lambda i: (0, 0)),
        ],
        out_specs=[
            pl.BlockSpec((BLK, 16), lambda i: (i, 0)),
            pl.BlockSpec((BLK, 16), lambda i: (i, 0)),
            pl.BlockSpec((2, BLK, 1), lambda i: (0, i, 0)),
        ],
        out_shape=[
            jax.ShapeDtypeStruct((NPAD, 16), _f32),
            jax.ShapeDtypeStruct((NPAD, 16), _f32),
            jax.ShapeDtypeStruct((2, NPAD, 1), _f32),
        ],
    )(hist4, xp, xl, wp, wl)


@functools.partial(
    pl.kernel,
    out_type=[
        jax.ShapeDtypeStruct((NC, NPAD, 16), _f32),
        jax.ShapeDtypeStruct((NC, NPAD, 16), _f32),
        jax.ShapeDtypeStruct((NC, NPAD), _f32),
        jax.ShapeDtypeStruct((NC, NPAD), _f32),
    ],
    mesh=_mesh,
    scratch_types=[
        pltpu.VMEM((NCH, CH), jnp.int32),
        pltpu.VMEM((NCH, CH), jnp.int32),
        pltpu.VMEM((CH, 16), _f32),
        pltpu.VMEM((CH, 16), _f32),
        pltpu.VMEM((NPAD,), _f32),
        pltpu.VMEM((NCH, CH), _f32),
        pltpu.VMEM_SHARED((NPAD, 16), _f32),
        pltpu.VMEM_SHARED((NPAD, 16), _f32),
        pltpu.VMEM_SHARED((NPAD,), _f32),
        pltpu.VMEM_SHARED((NPAD,), _f32),
        pltpu.SemaphoreType.DMA,
        pltpu.SemaphoreType.DMA,
        pltpu.SemaphoreType.DMA,
        pltpu.SemaphoreType.DMA,
        pltpu.SemaphoreType.DMA,
    ],
    compiler_params=_sc_params,
)
def _sc_edges(src_p, dst_p, src_l, dst_l, hp, hl, dvp, dvl, z16_hbm, z1_hbm,
              rop, rol, cop, col_,
              sidx2d, didx2d, rows0, rows1, dv_v, vals2d,
              racc_p, racc_l, cacc_p, cacc_l,
              sem_i, sem_g0, sem_g1, sem_s0, sem_s1):
    c = lax.axis_index("c")
    s = lax.axis_index("s")
    sl = pl.ds(s * SPT, SPT)
    pltpu.sync_copy(z16_hbm.at[sl], racc_p.at[sl])
    pltpu.sync_copy(z16_hbm.at[sl], racc_l.at[sl])
    pltpu.sync_copy(z1_hbm.at[sl], cacc_p.at[sl])
    pltpu.sync_copy(z1_hbm.at[sl], cacc_l.at[sl])
    plsc.subcore_barrier()
    base = (s * NC + c) * EPT
    rows_bufs = (rows0, rows1)
    gsems = (sem_g0, sem_g1)
    ssems = (sem_s0, sem_s1)
    for sref, dref, href, dvref, racc, cacc in (
        (src_p, dst_p, hp, dvp, racc_p, cacc_p),
        (src_l, dst_l, hl, dvl, racc_l, cacc_l),
    ):
        descs = [pltpu.async_copy(dvref, dv_v, sem_i)]
        for k in range(NCH):
            off = pl.ds(base + k * CH, CH)
            descs.append(pltpu.async_copy(sref.at[off], sidx2d.at[k], sem_i))
            descs.append(pltpu.async_copy(dref.at[off], didx2d.at[k], sem_i))
        for d in descs:
            d.wait()

        for k in range(NCH):
            def vbody(j, _, k=k):
                iv = didx2d[k, pl.ds(j * 16, 16)]
                vals2d[k, pl.ds(j * 16, 16)] = plsc.load_gather(dv_v, [iv])
                return 0
            lax.fori_loop(0, CH // 16, vbody, 0)

        gd = [None, None]
        sd = [None, None]
        gd[0] = pltpu.async_copy(href.at[sidx2d.at[0]], rows_bufs[0], gsems[0])
        for k in range(NCH):
            b = k % 2
            nb = (k + 1) % 2
            gd[b].wait()
            if k + 1 < NCH:
                if sd[nb] is not None:
                    sd[nb].wait()
                gd[nb] = pltpu.async_copy(href.at[sidx2d.at[k + 1]],
                                          rows_bufs[nb], gsems[nb])
            sd[b] = pltpu.async_copy(rows_bufs[b], racc.at[didx2d.at[k]],
                                     ssems[b], add=True)
            pltpu.sync_copy(vals2d.at[k], cacc.at[sidx2d.at[k]], add=True)
        for b in (0, 1):
            if sd[b] is not None:
                sd[b].wait()
    plsc.subcore_barrier()
    pltpu.sync_copy(racc_p.at[sl], rop.at[c].at[sl])
    pltpu.sync_copy(racc_l.at[sl], rol.at[c].at[sl])
    pltpu.sync_copy(cacc_p.at[sl], cop.at[c].at[sl])
    pltpu.sync_copy(cacc_l.at[sl], col_.at[c].at[sl])


def _tc2_body(rop_ref, rol_ref, cop_ref, col_ref, hp_ref, hl_ref, dinv_ref,
              bpin_ref, blin_ref, wpo_ref, wlo_ref, bpo_ref, blo_ref,
              w1_ref, b1_ref, w2_ref, b2_ref, w3_ref, b3_ref, act_ref,
              out_ref, sacc):
    i = pl.program_id(0)

    @pl.when(i == 0)
    def _():
        sacc[...] = jnp.zeros((8, 128), _f32)

    gid = lax.broadcasted_iota(jnp.int32, (BLK, 1), 0) + i * BLK
    valid = (gid < N).astype(_f32)

    for g, (ro_ref, co_ref, h_ref, b_ref) in enumerate((
        (rop_ref, cop_ref, hp_ref, bpin_ref),
        (rol_ref, col_ref, hl_ref, blin_ref),
    )):
        ro = ro_ref[...]
        dinv = dinv_ref[...][g]
        acc = ro[0] + ro[1] + h_ref[...]
        p1 = jax.nn.relu(dinv * acc + b_ref[...][None, :])
        co = co_ref[...]
        cvec = dinv * (co[0] + co[1] + dinv) * valid
        sacc[g, 0:16] += jnp.sum(cvec * p1, axis=0)

    @pl.when(i == GRID - 1)
    def _():
        inv_n = 1.0 / N
        sp = sacc[0:1, 0:16] * inv_n
        sl_ = sacc[1:2, 0:16] * inv_n
        pooled_p = jnp.dot(sp, wpo_ref[...], preferred_element_type=_f32) \
            + bpo_ref[...][None, :]
        pooled_l = jnp.dot(sl_, wlo_ref[...], preferred_element_type=_f32) \
            + blo_ref[...][None, :]
        m = jnp.concatenate([pooled_p, pooled_l], axis=1)
        fp = jax.nn.relu(jnp.dot(m, w1_ref[...],
                                 preferred_element_type=_f32)
                         + b1_ref[...][None, :])
        h2 = jnp.concatenate([fp, act_ref[...]], axis=1)
        pol = jax.nn.relu(jnp.dot(h2, w2_ref[...],
                                  preferred_element_type=_f32)
                          + b2_ref[...][None, :])
        out_ref[...] = jnp.dot(pol, w3_ref[...],
                               preferred_element_type=_f32) \
            + b3_ref[...][None, :]


def _tc2(rop, rol, cop4, col4, hp, hl, dinv, b_pin, b_lin, W_pout, W_lout,
         b_pout, b_lout, W1, b1, W2, b2, W3, b3, action):
    def full(shape):
        return pl.BlockSpec(shape, lambda *_: tuple(0 for _ in shape))
    return pl.pallas_call(
        _tc2_body,
        grid=(GRID,),
        in_specs=[
            pl.BlockSpec((NC, BLK, 16), lambda i: (0, i, 0)),
            pl.BlockSpec((NC, BLK, 16), lambda i: (0, i, 0)),
            pl.BlockSpec((NC, BLK, 1), lambda i: (0, i, 0)),
            pl.BlockSpec((NC, BLK, 1), lambda i: (0, i, 0)),
            pl.BlockSpec((BLK, 16), lambda i: (i, 0)),
            pl.BlockSpec((BLK, 16), lambda i: (i, 0)),
            pl.BlockSpec((2, BLK, 1), lambda i: (0, i, 0)),
            full((16,)), full((16,)),
            full((16, 50)), full((16, 50)),
            full((50,)), full((50,)),
            full((100, 60)), full((60,)),
            full((100, 10)), full((10,)),
            full((10, 1)), full((1,)),
            full((1, 40)),
        ],
        out_specs=pl.BlockSpec((1, 1), lambda i: (0, 0)),
        out_shape=jax.ShapeDtypeStruct((1, 1), _f32),
        scratch_shapes=[pltpu.VMEM((8, 128), _f32)],
    )(rop, rol, cop4, col4, hp, hl, dinv, b_pin, b_lin, W_pout, W_lout,
      b_pout, b_lout, W1, b1, W2, b2, W3, b3, action)


def kernel(protein_x, protein_edge_index, ligand_x, ligand_edge_index, action,
           W_pin, b_pin, W_pout, b_pout, W_lin, b_lin, W_lout, b_lout,
           W1, b1, W2, b2, W3, b3):
    z1 = jnp.zeros((NPAD,), _f32)
    z16 = jnp.zeros((NPAD, 16), _f32)
    ones = jnp.ones((CH,), _f32)
    xp = jnp.pad(protein_x, ((0, NPAD - N), (0, 0)))
    xl = jnp.pad(ligand_x, ((0, NPAD - N), (0, 0)))
    src_p, dst_p = protein_edge_index[0], protein_edge_index[1]
    src_l, dst_l = ligand_edge_index[0], ligand_edge_index[1]

    hist = _sc_hist(dst_p, dst_l, ones, z1)
    hp, hl, dinv = _tc1(hist.reshape(2, NC, NPAD, 1), xp, xl, W_pin, W_lin)
    dvp = dinv[0].reshape(NPAD)
    dvl = dinv[1].reshape(NPAD)
    rop, rol, cop, col_ = _sc_edges(src_p, dst_p, src_l, dst_l,
                                    hp, hl, dvp, dvl, z16, z1)
    return _tc2(rop, rol, cop.reshape(NC, NPAD, 1), col_.reshape(NC, NPAD, 1),
                hp, hl, dinv, b_pin, b_lin, W_pout, W_lout, b_pout, b_lout,
                W1, b1, W2, b2, W3, b3, action)


# speedup vs baseline: 73.1779x; 1.1147x over previous
import functools
import jax
import jax.numpy as jnp
from jax import lax
from jax.experimental import pallas as pl
from jax.experimental.pallas import tpu as pltpu
from jax.experimental.pallas import tpu_sc as plsc

N = 10000
E = 320000
D = 128
NPAD = 10240
NC = 2
NS = 16
NW = NC * NS
EPT = E // NW
CH = 2000
NCH = EPT // CH
SPT = NPAD // NS
BLK = 1024
GRID = NPAD // BLK

_mesh = plsc.VectorSubcoreMesh(core_axis_name="c", subcore_axis_name="s")
_f32 = jnp.float32
_sc_params = pltpu.CompilerParams(use_tc_tiling_on_sc=False,
                                  needs_layout_passes=False)


@functools.partial(
    pl.kernel,
    out_type=jax.ShapeDtypeStruct((2, NC, NPAD), _f32),
    mesh=_mesh,
    scratch_types=[
        pltpu.VMEM((EPT,), jnp.int32),
        pltpu.VMEM((EPT,), jnp.int32),
        pltpu.VMEM((EPT,), _f32),
        pltpu.VMEM_SHARED((NPAD,), _f32),
        pltpu.VMEM_SHARED((NPAD,), _f32),
        pltpu.SemaphoreType.DMA,
        pltpu.SemaphoreType.DMA,
    ],
    compiler_params=_sc_params,
)
def _sc_hist(dst_p, dst_l, ones_hbm, z1_hbm, hist_out, idx_p, idx_l, ones_v,
             acc_p, acc_l, sem, sem_s):
    c = lax.axis_index("c")
    s = lax.axis_index("s")
    sl = pl.ds(s * SPT, SPT)
    base = (s * NC + c) * EPT
    descs = [
        pltpu.async_copy(dst_p.at[pl.ds(base, EPT)], idx_p, sem),
        pltpu.async_copy(dst_l.at[pl.ds(base, EPT)], idx_l, sem),
        pltpu.async_copy(ones_hbm, ones_v, sem),
    ]
    pltpu.sync_copy(z1_hbm.at[sl], acc_p.at[sl])
    pltpu.sync_copy(z1_hbm.at[sl], acc_l.at[sl])
    for d in descs:
        d.wait()
    plsc.subcore_barrier()
    sp = pltpu.async_copy(ones_v, acc_p.at[idx_p], sem_s, add=True)
    sl2 = pltpu.async_copy(ones_v, acc_l.at[idx_l], sem_s, add=True)
    sp.wait()
    sl2.wait()
    plsc.subcore_barrier()
    pltpu.sync_copy(acc_p.at[sl], hist_out.at[0].at[c].at[sl])
    pltpu.sync_copy(acc_l.at[sl], hist_out.at[1].at[c].at[sl])


def _tc1_body(hist_ref, xp_ref, xl_ref, wp_ref, wl_ref, hp_ref, hl_ref, dinv_ref):
    hist = hist_ref[...]
    deg = hist[:, 0] + hist[:, 1] + 1.0
    dinv = lax.rsqrt(deg)
    dinv_ref[...] = dinv
    hp_ref[...] = jnp.dot(xp_ref[...], wp_ref[...],
                          preferred_element_type=_f32) * dinv[0]
    hl_ref[...] = jnp.dot(xl_ref[...], wl_ref[...],
                          preferred_element_type=_f32) * dinv[1]


def _tc1(hist4, xp, xl, wp, wl):
    return pl.pallas_call(
        _tc1_body,
        grid=(GRID,),
        in_specs=[
            pl.BlockSpec((2, NC, BLK, 1), lambda i: (0, 0, i, 0)),
            pl.BlockSpec((BLK, D), lambda i: (i, 0)),
            pl.BlockSpec((BLK, D), lambda i: (i, 0)),
            pl.BlockSpec((D, 16), lambda i: (0, 0)),
            pl.BlockSpec((D, 16), lambda i: (0, 0)),
        ],
        out_specs=[
            pl.BlockSpec((BLK, 16), lambda i: (i, 0)),
            pl.BlockSpec((BLK, 16), lambda i: (i, 0)),
            pl.BlockSpec((2, BLK, 1), lambda i: (0, i, 0)),
        ],
        out_shape=[
            jax.ShapeDtypeStruct((NPAD, 16), _f32),
            jax.ShapeDtypeStruct((NPAD, 16), _f32),
            jax.ShapeDtypeStruct((2, NPAD, 1), _f32),
        ],
    )(hist4, xp, xl, wp, wl)


@functools.partial(
    pl.kernel,
    out_type=[
        jax.ShapeDtypeStruct((NC, NPAD, 16), _f32),
        jax.ShapeDtypeStruct((NC, NPAD, 16), _f32),
        jax.ShapeDtypeStruct((NC, NPAD), _f32),
        jax.ShapeDtypeStruct((NC, NPAD), _f32),
    ],
    mesh=_mesh,
    scratch_types=[
        pltpu.VMEM((EPT,), jnp.int32),
        pltpu.VMEM((NCH, CH), jnp.int32),
        pltpu.VMEM((CH, 16), _f32),
        pltpu.VMEM((CH, 16), _f32),
        pltpu.VMEM((NPAD,), _f32),
        pltpu.VMEM((EPT,), _f32),
        pltpu.VMEM_SHARED((NPAD, 16), _f32),
        pltpu.VMEM_SHARED((NPAD, 16), _f32),
        pltpu.VMEM_SHARED((NPAD,), _f32),
        pltpu.VMEM_SHARED((NPAD,), _f32),
        pltpu.SemaphoreType.DMA,
        pltpu.SemaphoreType.DMA,
        pltpu.SemaphoreType.DMA,
        pltpu.SemaphoreType.DMA,
        pltpu.SemaphoreType.DMA,
        pltpu.SemaphoreType.DMA,
    ],
    compiler_params=_sc_params,
)
def _sc_edges(src_p, dst_p, src_l, dst_l, hp, hl, dvp, dvl, z16_hbm, z1_hbm,
              rop, rol, cop, col_,
              sidx, didx2d, rows0, rows1, dv_v, vals,
              racc_p, racc_l, cacc_p, cacc_l,
              sem_i, sem_g0, sem_g1, sem_s0, sem_s1, sem_c):
    c = lax.axis_index("c")
    s = lax.axis_index("s")
    sl = pl.ds(s * SPT, SPT)
    base = (s * NC + c) * EPT
    rows_bufs = (rows0, rows1)
    gsems = (sem_g0, sem_g1)
    ssems = (sem_s0, sem_s1)

    def fire_idx_loads(sref, dref, dvref):
        descs = [pltpu.async_copy(dvref, dv_v, sem_i),
                 pltpu.async_copy(sref.at[pl.ds(base, EPT)], sidx, sem_i)]
        for k in range(NCH):
            off = pl.ds(base + k * CH, CH)
            descs.append(pltpu.async_copy(dref.at[off], didx2d.at[k], sem_i))
        return descs

    graphs = (
        (src_p, dst_p, hp, dvp, racc_p, cacc_p),
        (src_l, dst_l, hl, dvl, racc_l, cacc_l),
    )
    descs = fire_idx_loads(graphs[0][0], graphs[0][1], graphs[0][3])
    pltpu.sync_copy(z16_hbm.at[sl], racc_p.at[sl])
    pltpu.sync_copy(z16_hbm.at[sl], racc_l.at[sl])
    pltpu.sync_copy(z1_hbm.at[sl], cacc_p.at[sl])
    pltpu.sync_copy(z1_hbm.at[sl], cacc_l.at[sl])
    plsc.subcore_barrier()
    cd = None
    for gi, (sref, dref, href, dvref, racc, cacc) in enumerate(graphs):
        if gi > 0:
            cd.wait()
            descs = fire_idx_loads(sref, dref, dvref)
        for d in descs:
            d.wait()

        for k in range(NCH):
            def vbody(j, _, k=k):
                iv = didx2d[k, pl.ds(j * 16, 16)]
                vals[pl.ds(k * CH + j * 16, 16)] = plsc.load_gather(dv_v, [iv])
                return 0
            lax.fori_loop(0, CH // 16, vbody, 0)
        cd = pltpu.async_copy(vals, cacc.at[sidx], sem_c, add=True)

        gd = [None, None]
        sd = [None, None]
        gd[0] = pltpu.async_copy(href.at[sidx.at[pl.ds(0, CH)]],
                                 rows_bufs[0], gsems[0])
        for k in range(NCH):
            b = k % 2
            nb = (k + 1) % 2
            gd[b].wait()
            if k + 1 < NCH:
                if sd[nb] is not None:
                    sd[nb].wait()
                gd[nb] = pltpu.async_copy(
                    href.at[sidx.at[pl.ds((k + 1) * CH, CH)]],
                    rows_bufs[nb], gsems[nb])
            sd[b] = pltpu.async_copy(rows_bufs[b], racc.at[didx2d.at[k]],
                                     ssems[b], add=True)
        for b in (0, 1):
            if sd[b] is not None:
                sd[b].wait()
    cd.wait()
    plsc.subcore_barrier()
    pltpu.sync_copy(racc_p.at[sl], rop.at[c].at[sl])
    pltpu.sync_copy(racc_l.at[sl], rol.at[c].at[sl])
    pltpu.sync_copy(cacc_p.at[sl], cop.at[c].at[sl])
    pltpu.sync_copy(cacc_l.at[sl], col_.at[c].at[sl])


def kernel(protein_x, protein_edge_index, ligand_x, ligand_edge_index, action,
           W_pin, b_pin, W_pout, b_pout, W_lin, b_lin, W_lout, b_lout,
           W1, b1, W2, b2, W3, b3):
    z1 = jnp.zeros((NPAD,), _f32)
    z16 = jnp.zeros((NPAD, 16), _f32)
    ones = jnp.ones((EPT,), _f32)
    xp = jnp.pad(protein_x, ((0, NPAD - N), (0, 0)))
    xl = jnp.pad(ligand_x, ((0, NPAD - N), (0, 0)))
    src_p, dst_p = protein_edge_index[0], protein_edge_index[1]
    src_l, dst_l = ligand_edge_index[0], ligand_edge_index[1]
    hist = _sc_hist(dst_p, dst_l, ones, z1)
    hp, hl, dinv = _tc1(hist.reshape(2, NC, NPAD, 1), xp, xl, W_pin, W_lin)
    dvp = dinv[0].reshape(NPAD)
    dvl = dinv[1].reshape(NPAD)
    rop, rol, cop, col_ = _sc_edges(src_p, dst_p, src_l, dst_l,
                                    hp, hl, dvp, dvl, z16, z1)

    def branch(h, dinvv, ro, co, bin_, Wout, bout):
        acc = ro[0][:N] + ro[1][:N] + h[:N]
        p1 = jax.nn.relu(dinvv[:N, None] * acc + bin_)
        c = dinvv[:N] * (co[0][:N] + co[1][:N] + dinvv[:N])
        s = c @ p1
        return (s @ Wout) / N + bout

    p = branch(hp, dvp, rop, cop, b_pin, W_pout, b_pout)
    l = branch(hl, dvl, rol, col_, b_lin, W_lout, b_lout)
    m = jnp.concatenate([p[None], l[None]], axis=1)
    fp = jax.nn.relu(m @ W1 + b1)
    pol = jnp.concatenate([fp, action], axis=1) @ W2 + b2
    return jax.nn.relu(pol) @ W3 + b3
